# Initial kernel scaffold; baseline (speedup 1.0000x reference)
import jax
import jax.numpy as jnp
from jax.experimental import pallas as pl


def kernel(pred_logits, pos_neg_probs, tgt_labels):
    raise NotImplementedError("write your pallas kernel here")


# speedup vs baseline: 3.3154x; 3.3154x over previous
import jax
import jax.numpy as jnp
from jax import lax
from jax.experimental import pallas as pl
from jax.experimental.pallas import tpu as pltpu

B, Q, C, T, L = 16, 4096, 128, 64, 6
BQ = 512
NQ = Q // BQ


def _body(logits_ref, pn_ref, tgt_ref, bq_out, keep_out, ms_out, bv_s, bi_s):
    qi = pl.program_id(1)

    @pl.when(qi == 0)
    def _init():
        bv_s[...] = jnp.full((1, C), -jnp.inf, jnp.float32)
        bi_s[...] = jnp.zeros((1, C), jnp.int32)

    x = logits_ref[0]
    xm = jnp.max(x, axis=1, keepdims=True)
    e = jnp.exp(x - xm)
    s = jnp.sum(e, axis=1, keepdims=True)
    p = e / s
    bm = jnp.max(p, axis=0, keepdims=True)
    ri = lax.broadcasted_iota(jnp.int32, (BQ, C), 0)
    cand = jnp.where(p == bm, ri, BQ)
    bidx = jnp.min(cand, axis=0, keepdims=True) + qi * BQ
    upd = bm > bv_s[...]
    bi_s[...] = jnp.where(upd, bidx, bi_s[...])
    bv_s[...] = jnp.where(upd, bm, bv_s[...])

    @pl.when(qi == NQ - 1)
    def _final():
        avg = jnp.mean(pn_ref[0], axis=0)
        lbl = tgt_ref[0]
        cls = lax.broadcasted_iota(jnp.int32, (T, C), 1)
        ohc = lbl == cls
        bif = jnp.broadcast_to(bi_s[...], (T, C))
        q_star = jnp.sum(jnp.where(ohc, bif, 0), axis=1, keepdims=True)
        r = q_star // C
        l = q_star % C
        ohr = r.reshape(T, 1, 1) == lax.broadcasted_iota(jnp.int32, (T, Q // C, C), 1)
        ohl = l.reshape(T, 1, 1) == lax.broadcasted_iota(jnp.int32, (T, Q // C, C), 2)
        picked = jnp.where(ohr & ohl, jnp.broadcast_to(avg[None], (T, Q // C, C)), 0.0)
        ms = jnp.sum(jnp.sum(picked, axis=2), axis=1, keepdims=True)
        mask = (ms > 0.3).astype(jnp.int32)
        anyh = jnp.sum(mask) > 0
        keep = jnp.where(anyh, mask, jnp.ones_like(mask))
        bq_out[0] = q_star
        keep_out[0] = keep
        ms_out[0] = ms


def kernel(pred_logits, pos_neg_probs, tgt_labels):
    pn = pos_neg_probs[..., 1]
    pn_t = jnp.transpose(pn, (1, 0, 2)).reshape(B, L, Q // C, C)
    tgt3 = tgt_labels.reshape(B, T, 1).astype(jnp.int32)

    grid = (B, NQ)
    out = pl.pallas_call(
        _body,
        grid=grid,
        in_specs=[
            pl.BlockSpec((1, BQ, C), lambda b, qi: (b, qi, 0)),
            pl.BlockSpec((1, L, Q // C, C), lambda b, qi: (b, 0, 0, 0)),
            pl.BlockSpec((1, T, 1), lambda b, qi: (b, 0, 0)),
        ],
        out_specs=[
            pl.BlockSpec((1, T, 1), lambda b, qi: (b, 0, 0)),
            pl.BlockSpec((1, T, 1), lambda b, qi: (b, 0, 0)),
            pl.BlockSpec((1, T, 1), lambda b, qi: (b, 0, 0)),
        ],
        out_shape=[
            jax.ShapeDtypeStruct((B, T, 1), jnp.int32),
            jax.ShapeDtypeStruct((B, T, 1), jnp.int32),
            jax.ShapeDtypeStruct((B, T, 1), jnp.float32),
        ],
        scratch_shapes=[
            pltpu.VMEM((1, C), jnp.float32),
            pltpu.VMEM((1, C), jnp.int32),
        ],
    )(pred_logits, pn_t, tgt3)

    bq, keep, ms = out
    base_query_idx = bq.reshape(B, T)
    base_target_idx = jnp.broadcast_to(jnp.arange(T, dtype=tgt_labels.dtype)[None, :], (B, T))
    keep_mask = keep.reshape(B, T).astype(jnp.bool_)
    matched_scores = ms.reshape(B, T)
    return (base_query_idx, base_target_idx, keep_mask, matched_scores)


# speedup vs baseline: 4.7305x; 1.4268x over previous
import jax
import jax.numpy as jnp
from jax import lax
from jax.experimental import pallas as pl
from jax.experimental.pallas import tpu as pltpu

B, Q, C, T, L = 16, 4096, 128, 64, 6
BQ = 1024
NQ = Q // BQ


def _body(logits_ref, pn_ref, tgt_ref, bq_out, keep_out, ms_out, bv_s, bi_s):
    qi = pl.program_id(1)

    @pl.when(qi == 0)
    def _init():
        bv_s[...] = jnp.full((1, C), -jnp.inf, jnp.float32)
        bi_s[...] = jnp.zeros((1, C), jnp.int32)

    x = logits_ref[0]
    xm = jnp.max(x, axis=1, keepdims=True)
    e = jnp.exp(x - xm)
    s = jnp.sum(e, axis=1, keepdims=True)
    p = e / s
    bm = jnp.max(p, axis=0, keepdims=True)
    ri = lax.broadcasted_iota(jnp.int32, (BQ, C), 0)
    cand = jnp.where(p == bm, ri, BQ)
    bidx = jnp.min(cand, axis=0, keepdims=True) + qi * BQ
    upd = bm > bv_s[...]
    bi_s[...] = jnp.where(upd, bidx, bi_s[...])
    bv_s[...] = jnp.where(upd, bm, bv_s[...])

    @pl.when(qi == NQ - 1)
    def _final():
        avg = jnp.mean(pn_ref[0], axis=0)
        lbl = tgt_ref[0]
        cls = lax.broadcasted_iota(jnp.int32, (T, C), 1)
        ohc = lbl == cls
        bif = jnp.broadcast_to(bi_s[...], (T, C))
        q_star = jnp.sum(jnp.where(ohc, bif, 0), axis=1, keepdims=True)
        flat = (lax.broadcasted_iota(jnp.int32, (T, Q // C, C), 1) * C
                + lax.broadcasted_iota(jnp.int32, (T, Q // C, C), 2))
        ohf = flat == q_star.reshape(T, 1, 1)
        picked = jnp.where(ohf, jnp.broadcast_to(avg[None], (T, Q // C, C)), 0.0)
        ms = jnp.sum(jnp.sum(picked, axis=2), axis=1, keepdims=True)
        mask = (ms > 0.3).astype(jnp.int32)
        anyh = jnp.sum(mask) > 0
        keep = jnp.where(anyh, mask, jnp.ones_like(mask))
        bq_out[0] = q_star
        keep_out[0] = keep
        ms_out[0] = ms


def kernel(pred_logits, pos_neg_probs, tgt_labels):
    pn = pos_neg_probs[..., 1]
    pn_t = jnp.transpose(pn, (1, 0, 2)).reshape(B, L, Q // C, C)
    tgt3 = tgt_labels.reshape(B, T, 1).astype(jnp.int32)

    grid = (B, NQ)
    out = pl.pallas_call(
        _body,
        grid=grid,
        in_specs=[
            pl.BlockSpec((1, BQ, C), lambda b, qi: (b, qi, 0)),
            pl.BlockSpec((1, L, Q // C, C), lambda b, qi: (b, 0, 0, 0)),
            pl.BlockSpec((1, T, 1), lambda b, qi: (b, 0, 0)),
        ],
        out_specs=[
            pl.BlockSpec((1, T, 1), lambda b, qi: (b, 0, 0)),
            pl.BlockSpec((1, T, 1), lambda b, qi: (b, 0, 0)),
            pl.BlockSpec((1, T, 1), lambda b, qi: (b, 0, 0)),
        ],
        out_shape=[
            jax.ShapeDtypeStruct((B, T, 1), jnp.int32),
            jax.ShapeDtypeStruct((B, T, 1), jnp.int32),
            jax.ShapeDtypeStruct((B, T, 1), jnp.float32),
        ],
        scratch_shapes=[
            pltpu.VMEM((1, C), jnp.float32),
            pltpu.VMEM((1, C), jnp.int32),
        ],
    )(pred_logits, pn_t, tgt3)

    bq, keep, ms = out
    base_query_idx = bq.reshape(B, T)
    base_target_idx = jnp.broadcast_to(jnp.arange(T, dtype=tgt_labels.dtype)[None, :], (B, T))
    keep_mask = keep.reshape(B, T).astype(jnp.bool_)
    matched_scores = ms.reshape(B, T)
    return (base_query_idx, base_target_idx, keep_mask, matched_scores)


# speedup vs baseline: 5.9388x; 1.2554x over previous
import jax
import jax.numpy as jnp
from jax import lax
from jax.experimental import pallas as pl
from jax.experimental.pallas import tpu as pltpu

B, Q, C, T, L = 16, 4096, 128, 64, 6
BQ = 2048
NQ = Q // BQ


def _body(logits_ref, pn_ref, tgt_ref, bq_out, keep_out, ms_out, bv_s, bi_s):
    qi = pl.program_id(1)

    @pl.when(qi == 0)
    def _init():
        bv_s[...] = jnp.full((1, C), -jnp.inf, jnp.float32)
        bi_s[...] = jnp.zeros((1, C), jnp.int32)

    x = logits_ref[0]
    xm = jnp.max(x, axis=1, keepdims=True)
    e = jnp.exp(x - xm)
    s = jnp.sum(e, axis=1, keepdims=True)
    p = e / s
    bm = jnp.max(p, axis=0, keepdims=True)
    ri = lax.broadcasted_iota(jnp.int32, (BQ, C), 0)
    cand = jnp.where(p == bm, ri, BQ)
    bidx = jnp.min(cand, axis=0, keepdims=True) + qi * BQ
    upd = bm > bv_s[...]
    bi_s[...] = jnp.where(upd, bidx, bi_s[...])
    bv_s[...] = jnp.where(upd, bm, bv_s[...])

    @pl.when(qi == NQ - 1)
    def _final():
        avg = jnp.mean(pn_ref[0], axis=0)
        lbl = tgt_ref[0]
        cls = lax.broadcasted_iota(jnp.int32, (T, C), 1)
        ohc = lbl == cls
        bif = jnp.broadcast_to(bi_s[...], (T, C))
        q_star = jnp.sum(jnp.where(ohc, bif, 0), axis=1, keepdims=True)
        flat = (lax.broadcasted_iota(jnp.int32, (T, Q // C, C), 1) * C
                + lax.broadcasted_iota(jnp.int32, (T, Q // C, C), 2))
        ohf = flat == q_star.reshape(T, 1, 1)
        picked = jnp.where(ohf, jnp.broadcast_to(avg[None], (T, Q // C, C)), 0.0)
        ms = jnp.sum(jnp.sum(picked, axis=2), axis=1, keepdims=True)
        mask = (ms > 0.3).astype(jnp.int32)
        anyh = jnp.sum(mask) > 0
        keep = jnp.where(anyh, mask, jnp.ones_like(mask))
        bq_out[0] = q_star
        keep_out[0] = keep
        ms_out[0] = ms


def kernel(pred_logits, pos_neg_probs, tgt_labels):
    pn = pos_neg_probs[..., 1]
    pn_t = jnp.transpose(pn, (1, 0, 2)).reshape(B, L, Q // C, C)
    tgt3 = tgt_labels.reshape(B, T, 1).astype(jnp.int32)

    grid = (B, NQ)
    out = pl.pallas_call(
        _body,
        grid=grid,
        in_specs=[
            pl.BlockSpec((1, BQ, C), lambda b, qi: (b, qi, 0)),
            pl.BlockSpec((1, L, Q // C, C), lambda b, qi: (b, 0, 0, 0)),
            pl.BlockSpec((1, T, 1), lambda b, qi: (b, 0, 0)),
        ],
        out_specs=[
            pl.BlockSpec((1, T, 1), lambda b, qi: (b, 0, 0)),
            pl.BlockSpec((1, T, 1), lambda b, qi: (b, 0, 0)),
            pl.BlockSpec((1, T, 1), lambda b, qi: (b, 0, 0)),
        ],
        out_shape=[
            jax.ShapeDtypeStruct((B, T, 1), jnp.int32),
            jax.ShapeDtypeStruct((B, T, 1), jnp.int32),
            jax.ShapeDtypeStruct((B, T, 1), jnp.float32),
        ],
        scratch_shapes=[
            pltpu.VMEM((1, C), jnp.float32),
            pltpu.VMEM((1, C), jnp.int32),
        ],
    )(pred_logits, pn_t, tgt3)

    bq, keep, ms = out
    base_query_idx = bq.reshape(B, T)
    base_target_idx = jnp.broadcast_to(jnp.arange(T, dtype=tgt_labels.dtype)[None, :], (B, T))
    keep_mask = keep.reshape(B, T).astype(jnp.bool_)
    matched_scores = ms.reshape(B, T)
    return (base_query_idx, base_target_idx, keep_mask, matched_scores)


# speedup vs baseline: 6.7123x; 1.1303x over previous
import jax
import jax.numpy as jnp
from jax import lax
from jax.experimental import pallas as pl
from jax.experimental.pallas import tpu as pltpu

B, Q, C, T, L = 16, 4096, 128, 64, 6
BQ = 4096
NQ = Q // BQ


def _body(logits_ref, pn_ref, tgt_ref, bq_out, keep_out, ms_out, bv_s, bi_s):
    qi = pl.program_id(1)

    @pl.when(qi == 0)
    def _init():
        bv_s[...] = jnp.full((1, C), -jnp.inf, jnp.float32)
        bi_s[...] = jnp.zeros((1, C), jnp.int32)

    x = logits_ref[0]
    xm = jnp.max(x, axis=1, keepdims=True)
    e = jnp.exp(x - xm)
    s = jnp.sum(e, axis=1, keepdims=True)
    p = e / s
    bm = jnp.max(p, axis=0, keepdims=True)
    ri = lax.broadcasted_iota(jnp.int32, (BQ, C), 0)
    cand = jnp.where(p == bm, ri, BQ)
    bidx = jnp.min(cand, axis=0, keepdims=True) + qi * BQ
    upd = bm > bv_s[...]
    bi_s[...] = jnp.where(upd, bidx, bi_s[...])
    bv_s[...] = jnp.where(upd, bm, bv_s[...])

    @pl.when(qi == NQ - 1)
    def _final():
        avg = jnp.mean(pn_ref[0], axis=0)
        lbl = tgt_ref[0]
        cls = lax.broadcasted_iota(jnp.int32, (T, C), 1)
        ohc = lbl == cls
        bif = jnp.broadcast_to(bi_s[...], (T, C))
        q_star = jnp.sum(jnp.where(ohc, bif, 0), axis=1, keepdims=True)
        flat = (lax.broadcasted_iota(jnp.int32, (T, Q // C, C), 1) * C
                + lax.broadcasted_iota(jnp.int32, (T, Q // C, C), 2))
        ohf = flat == q_star.reshape(T, 1, 1)
        picked = jnp.where(ohf, jnp.broadcast_to(avg[None], (T, Q // C, C)), 0.0)
        ms = jnp.sum(jnp.sum(picked, axis=2), axis=1, keepdims=True)
        mask = (ms > 0.3).astype(jnp.int32)
        anyh = jnp.sum(mask) > 0
        keep = jnp.where(anyh, mask, jnp.ones_like(mask))
        bq_out[0] = q_star
        keep_out[0] = keep
        ms_out[0] = ms


def kernel(pred_logits, pos_neg_probs, tgt_labels):
    pn = pos_neg_probs[..., 1]
    pn_t = jnp.transpose(pn, (1, 0, 2)).reshape(B, L, Q // C, C)
    tgt3 = tgt_labels.reshape(B, T, 1).astype(jnp.int32)

    grid = (B, NQ)
    out = pl.pallas_call(
        _body,
        grid=grid,
        in_specs=[
            pl.BlockSpec((1, BQ, C), lambda b, qi: (b, qi, 0)),
            pl.BlockSpec((1, L, Q // C, C), lambda b, qi: (b, 0, 0, 0)),
            pl.BlockSpec((1, T, 1), lambda b, qi: (b, 0, 0)),
        ],
        out_specs=[
            pl.BlockSpec((1, T, 1), lambda b, qi: (b, 0, 0)),
            pl.BlockSpec((1, T, 1), lambda b, qi: (b, 0, 0)),
            pl.BlockSpec((1, T, 1), lambda b, qi: (b, 0, 0)),
        ],
        out_shape=[
            jax.ShapeDtypeStruct((B, T, 1), jnp.int32),
            jax.ShapeDtypeStruct((B, T, 1), jnp.int32),
            jax.ShapeDtypeStruct((B, T, 1), jnp.float32),
        ],
        scratch_shapes=[
            pltpu.VMEM((1, C), jnp.float32),
            pltpu.VMEM((1, C), jnp.int32),
        ],
    )(pred_logits, pn_t, tgt3)

    bq, keep, ms = out
    base_query_idx = bq.reshape(B, T)
    base_target_idx = jnp.broadcast_to(jnp.arange(T, dtype=tgt_labels.dtype)[None, :], (B, T))
    keep_mask = keep.reshape(B, T).astype(jnp.bool_)
    matched_scores = ms.reshape(B, T)
    return (base_query_idx, base_target_idx, keep_mask, matched_scores)
